# Initial kernel scaffold; baseline (speedup 1.0000x reference)
import jax
import jax.numpy as jnp
from jax.experimental import pallas as pl


def kernel(ids_0, ids_1, table_0, table_1):
    raise NotImplementedError("write your pallas kernel here")


# speedup vs baseline: 28.9104x; 28.9104x over previous
import functools

import jax
import jax.numpy as jnp
from jax import lax
from jax.experimental import pallas as pl
from jax.experimental.pallas import tpu as pltpu
from jax.experimental.pallas import tpu_sc as plsc

B, L, DIM = 16384, 20, 128
ZCH_0, ZCH_1 = 8, 16

NC, NS, LANES = 2, 16, 16
NW = NC * NS
BAGS_PER_TILE = B // NW
GROUPS = BAGS_PER_TILE // LANES

BLK = 2048


def _hist_body(ids0_hbm, ids1_hbm, c0_hbm, c1_hbm, ids0_v, ids1_v, c0_v, c1_v):
    wid = lax.axis_index("s") * NC + lax.axis_index("c")
    base = wid * BAGS_PER_TILE

    pltpu.sync_copy(ids0_hbm.at[pl.ds(base * L, BAGS_PER_TILE * L)], ids0_v)
    pltpu.sync_copy(ids1_hbm.at[pl.ds(base * L, BAGS_PER_TILE * L)], ids1_v)

    iota = lax.iota(jnp.int32, LANES)
    zeros = jnp.zeros((LANES,), jnp.float32)
    ones = zeros + 1.0

    def zero0(i, _):
        c0_v[pl.ds(i * LANES, LANES)] = zeros
        return 0

    def zero1(i, _):
        c1_v[pl.ds(i * LANES, LANES)] = zeros
        return 0

    lax.fori_loop(0, BAGS_PER_TILE * ZCH_0 // LANES, zero0, 0)
    lax.fori_loop(0, BAGS_PER_TILE * ZCH_1 // LANES, zero1, 0)

    def group(g, _):
        lbag = g * LANES + iota
        pos = lbag * L
        cb0 = lbag * ZCH_0
        cb1 = lbag * ZCH_1
        for l in range(L):
            g0 = plsc.load_gather(ids0_v, [pos + l])
            g1 = plsc.load_gather(ids1_v, [pos + l])
            e0 = g0 & (ZCH_0 - 1)
            e1 = g1 & (ZCH_1 - 1)
            plsc.addupdate_scatter(c0_v, [cb0 + e0], ones)
            plsc.addupdate_scatter(c1_v, [cb1 + e1], ones)
        return 0

    lax.fori_loop(0, GROUPS, group, 0)

    pltpu.sync_copy(c0_v, c0_hbm.at[pl.ds(base * ZCH_0, BAGS_PER_TILE * ZCH_0)])
    pltpu.sync_copy(c1_v, c1_hbm.at[pl.ds(base * ZCH_1, BAGS_PER_TILE * ZCH_1)])


@functools.lru_cache(maxsize=None)
def _hist():
    return pl.kernel(
        _hist_body,
        out_type=(
            jax.ShapeDtypeStruct((B * ZCH_0,), jnp.float32),
            jax.ShapeDtypeStruct((B * ZCH_1,), jnp.float32),
        ),
        mesh=plsc.VectorSubcoreMesh(
            core_axis_name="c", subcore_axis_name="s", num_cores=NC, num_subcores=NS
        ),
        compiler_params=pltpu.CompilerParams(needs_layout_passes=False),
        scratch_types=[
            pltpu.VMEM((BAGS_PER_TILE * L,), jnp.int32),
            pltpu.VMEM((BAGS_PER_TILE * L,), jnp.int32),
            pltpu.VMEM((BAGS_PER_TILE * ZCH_0,), jnp.float32),
            pltpu.VMEM((BAGS_PER_TILE * ZCH_1,), jnp.float32),
        ],
    )


def _mm_body(c0_ref, c1_ref, t0_ref, t1_ref, out_ref, loss_ref):
    i = pl.program_id(0)
    p0 = jnp.dot(c0_ref[...], t0_ref[...], preferred_element_type=jnp.float32,
                 precision=lax.Precision.HIGHEST)
    p1 = jnp.dot(c1_ref[...], t1_ref[...], preferred_element_type=jnp.float32,
                 precision=lax.Precision.HIGHEST)
    out_ref[...] = jnp.concatenate([p0, p1], axis=1)

    @pl.when(i == 0)
    def _():
        loss_ref[0, 0] = 0.0

    loss_ref[0, 0] += jnp.sum(p0) + jnp.sum(p1)


def _pool_matmul(c0, c1, table_0, table_1):
    return pl.pallas_call(
        _mm_body,
        grid=(B // BLK,),
        in_specs=[
            pl.BlockSpec((BLK, ZCH_0), lambda i: (i, 0)),
            pl.BlockSpec((BLK, ZCH_1), lambda i: (i, 0)),
            pl.BlockSpec((ZCH_0, DIM), lambda i: (0, 0)),
            pl.BlockSpec((ZCH_1, DIM), lambda i: (0, 0)),
        ],
        out_specs=[
            pl.BlockSpec((BLK, 2 * DIM), lambda i: (i, 0)),
            pl.BlockSpec((1, 1), lambda i: (0, 0), memory_space=pltpu.SMEM),
        ],
        out_shape=[
            jax.ShapeDtypeStruct((B, 2 * DIM), jnp.float32),
            jax.ShapeDtypeStruct((1, 1), jnp.float32),
        ],
    )(c0, c1, table_0, table_1)


@jax.jit
def kernel(ids_0, ids_1, table_0, table_1):
    ids0_flat = ids_0.astype(jnp.int32).reshape(-1)
    ids1_flat = ids_1.astype(jnp.int32).reshape(-1)
    c0_flat, c1_flat = _hist()(ids0_flat, ids1_flat)
    c0 = c0_flat.reshape(B, ZCH_0)
    c1 = c1_flat.reshape(B, ZCH_1)
    pred, loss_sum = _pool_matmul(c0, c1, table_0[:ZCH_0], table_1[:ZCH_1])
    loss = loss_sum[0, 0] / (B * 2 * DIM)
    return (loss, pred)


# speedup vs baseline: 33.6948x; 1.1655x over previous
import functools

import jax
import jax.numpy as jnp
from jax import lax
from jax.experimental import pallas as pl
from jax.experimental.pallas import tpu as pltpu
from jax.experimental.pallas import tpu_sc as plsc

B, L, DIM = 16384, 20, 128
ZCH_0, ZCH_1 = 8, 16
NBINS = ZCH_0 + ZCH_1

NC, NS, LANES = 2, 16, 16
NW = NC * NS
BPT = B // NW
GROUPS = BPT // LANES
CSZ = BPT * NBINS

TPB = 4
BLK = TPB * BPT


def _hist_body(ids0_hbm, ids1_hbm, cnt_hbm, ids0_v, ids1_v, cnt_v):
    wid = lax.axis_index("s") * NC + lax.axis_index("c")
    base = wid * BPT

    pltpu.sync_copy(ids0_hbm.at[pl.ds(base * L, BPT * L)], ids0_v)
    pltpu.sync_copy(ids1_hbm.at[pl.ds(base * L, BPT * L)], ids1_v)

    iota = lax.iota(jnp.int32, LANES)
    zeros = jnp.zeros((LANES,), jnp.float32)
    ones = zeros + 1.0

    def zinit(i, _):
        for j in range(16):
            cnt_v[pl.ds((i * 16 + j) * LANES, LANES)] = zeros
        return 0

    lax.fori_loop(0, CSZ // (16 * LANES), zinit, 0)

    def group(g, _):
        lbag = g * LANES + iota
        pos = lbag * L
        row0 = lbag * NBINS
        row1 = row0 + ZCH_0
        for l in range(L):
            g0 = plsc.load_gather(ids0_v, [pos + l])
            g1 = plsc.load_gather(ids1_v, [pos + l])
            e0 = g0 & (ZCH_0 - 1)
            e1 = g1 & (ZCH_1 - 1)
            plsc.addupdate_scatter(cnt_v, [row0 + e0], ones)
            plsc.addupdate_scatter(cnt_v, [row1 + e1], ones)
        return 0

    lax.fori_loop(0, GROUPS, group, 0)

    pltpu.sync_copy(cnt_v, cnt_hbm.at[pl.ds(wid * CSZ, CSZ)])


@functools.lru_cache(maxsize=None)
def _hist():
    return pl.kernel(
        _hist_body,
        out_type=jax.ShapeDtypeStruct((NW * CSZ,), jnp.float32),
        mesh=plsc.VectorSubcoreMesh(
            core_axis_name="c", subcore_axis_name="s", num_cores=NC, num_subcores=NS
        ),
        compiler_params=pltpu.CompilerParams(needs_layout_passes=False),
        scratch_types=[
            pltpu.VMEM((BPT * L,), jnp.int32),
            pltpu.VMEM((BPT * L,), jnp.int32),
            pltpu.VMEM((CSZ,), jnp.float32),
        ],
    )


def _mm_body(ct_ref, t0_ref, t1_ref, out_ref, loss_ref):
    i = pl.program_id(0)
    ct = ct_ref[...].reshape(BLK, NBINS)
    zz = jnp.zeros((ZCH_0, DIM), jnp.float32)
    w = jnp.concatenate(
        [
            jnp.concatenate([t0_ref[...], zz], axis=1),
            jnp.concatenate([jnp.zeros((ZCH_1, DIM), jnp.float32), t1_ref[...]], axis=1),
        ],
        axis=0,
    )
    ct_bf = ct.astype(jnp.bfloat16)
    w_hi = w.astype(jnp.bfloat16)
    w_lo = (w - w_hi.astype(jnp.float32)).astype(jnp.bfloat16)
    p = jnp.dot(ct_bf, w_hi, preferred_element_type=jnp.float32)
    p = p + jnp.dot(ct_bf, w_lo, preferred_element_type=jnp.float32)
    out_ref[...] = p

    @pl.when(i == 0)
    def _():
        loss_ref[0, 0] = 0.0

    loss_ref[0, 0] += jnp.sum(p)


def _pool_matmul(ct, table_0, table_1):
    return pl.pallas_call(
        _mm_body,
        grid=(B // BLK,),
        in_specs=[
            pl.BlockSpec((TPB, BPT, NBINS), lambda i: (i, 0, 0)),
            pl.BlockSpec((ZCH_0, DIM), lambda i: (0, 0)),
            pl.BlockSpec((ZCH_1, DIM), lambda i: (0, 0)),
        ],
        out_specs=[
            pl.BlockSpec((BLK, 2 * DIM), lambda i: (i, 0)),
            pl.BlockSpec((1, 1), lambda i: (0, 0), memory_space=pltpu.SMEM),
        ],
        out_shape=[
            jax.ShapeDtypeStruct((B, 2 * DIM), jnp.float32),
            jax.ShapeDtypeStruct((1, 1), jnp.float32),
        ],
    )(ct, table_0, table_1)


@jax.jit
def kernel(ids_0, ids_1, table_0, table_1):
    ids0_flat = ids_0.astype(jnp.int32).reshape(-1)
    ids1_flat = ids_1.astype(jnp.int32).reshape(-1)
    cnt_flat = _hist()(ids0_flat, ids1_flat)
    ct = cnt_flat.reshape(NW, BPT, NBINS)
    pred, loss_sum = _pool_matmul(ct, table_0[:ZCH_0], table_1[:ZCH_1])
    loss = loss_sum[0, 0] / (B * 2 * DIM)
    return (loss, pred)


# speedup vs baseline: 36.6369x; 1.0873x over previous
import functools

import jax
import jax.numpy as jnp
from jax import lax
from jax.experimental import pallas as pl
from jax.experimental.pallas import tpu as pltpu
from jax.experimental.pallas import tpu_sc as plsc

B, L, DIM = 16384, 20, 128
ZCH_0, ZCH_1 = 8, 16
NBINS = ZCH_0 + ZCH_1

NC, NS, LANES = 2, 16, 16
NW = NC * NS
BPT = B // NW
GROUPS = BPT // LANES

BLK = 2048


def _hist_body(ids0_hbm, ids1_hbm, cnt_hbm, ids_v, cnt_v):
    wid = lax.axis_index("s") * NC + lax.axis_index("c")
    base = wid * BPT

    iota = lax.iota(jnp.int32, LANES)
    zeros = jnp.zeros((LANES,), jnp.float32)
    ones = zeros + 1.0

    def zrow(i, _):
        r = i // (BPT // LANES)
        c = i % (BPT // LANES)
        cnt_v[r, pl.ds(c * LANES, LANES)] = zeros
        return 0

    lax.fori_loop(0, NBINS * (BPT // LANES), zrow, 0)

    def run_feature(ids_hbm, zch, row_off):
        pltpu.sync_copy(ids_hbm.at[pl.ds(base, BPT), :], ids_v)

        def group(g, _):
            lbag = g * LANES + iota
            for l in range(L):
                lvec = iota * 0 + l
                gid = plsc.load_gather(ids_v, [lbag, lvec])
                e = (gid & (zch - 1)) + row_off
                plsc.addupdate_scatter(cnt_v, [e, lbag], ones)
            return 0

        lax.fori_loop(0, GROUPS, group, 0)

    run_feature(ids0_hbm, ZCH_0, 0)
    run_feature(ids1_hbm, ZCH_1, ZCH_0)

    pltpu.sync_copy(cnt_v, cnt_hbm.at[:, pl.ds(base, BPT)])


@functools.lru_cache(maxsize=None)
def _hist():
    return pl.kernel(
        _hist_body,
        out_type=jax.ShapeDtypeStruct((NBINS, B), jnp.float32),
        mesh=plsc.VectorSubcoreMesh(
            core_axis_name="c", subcore_axis_name="s", num_cores=NC, num_subcores=NS
        ),
        compiler_params=pltpu.CompilerParams(needs_layout_passes=False),
        scratch_types=[
            pltpu.VMEM((BPT, L), jnp.int32),
            pltpu.VMEM((NBINS, BPT), jnp.float32),
        ],
    )


def _mm_body(ct_ref, t0_ref, t1_ref, out_ref, loss_ref):
    i = pl.program_id(0)
    ct = ct_ref[...]
    zz = jnp.zeros((ZCH_0, DIM), jnp.float32)
    w = jnp.concatenate(
        [
            jnp.concatenate([t0_ref[...], zz], axis=1),
            jnp.concatenate([jnp.zeros((ZCH_1, DIM), jnp.float32), t1_ref[...]], axis=1),
        ],
        axis=0,
    )
    ct_bf = ct.astype(jnp.bfloat16)
    w_hi = w.astype(jnp.bfloat16)
    w_lo = (w - w_hi.astype(jnp.float32)).astype(jnp.bfloat16)
    dn = (((0,), (0,)), ((), ()))
    p = lax.dot_general(ct_bf, w_hi, dn, preferred_element_type=jnp.float32)
    p = p + lax.dot_general(ct_bf, w_lo, dn, preferred_element_type=jnp.float32)
    out_ref[...] = p

    @pl.when(i == 0)
    def _():
        loss_ref[0, 0] = 0.0

    loss_ref[0, 0] += jnp.sum(p)


def _pool_matmul(ct, table_0, table_1):
    return pl.pallas_call(
        _mm_body,
        grid=(B // BLK,),
        in_specs=[
            pl.BlockSpec((NBINS, BLK), lambda i: (0, i)),
            pl.BlockSpec((ZCH_0, DIM), lambda i: (0, 0)),
            pl.BlockSpec((ZCH_1, DIM), lambda i: (0, 0)),
        ],
        out_specs=[
            pl.BlockSpec((BLK, 2 * DIM), lambda i: (i, 0)),
            pl.BlockSpec((1, 1), lambda i: (0, 0), memory_space=pltpu.SMEM),
        ],
        out_shape=[
            jax.ShapeDtypeStruct((B, 2 * DIM), jnp.float32),
            jax.ShapeDtypeStruct((1, 1), jnp.float32),
        ],
    )(ct, table_0, table_1)


@jax.jit
def kernel(ids_0, ids_1, table_0, table_1):
    ct = _hist()(ids_0.astype(jnp.int32), ids_1.astype(jnp.int32))
    pred, loss_sum = _pool_matmul(ct, table_0[:ZCH_0], table_1[:ZCH_1])
    loss = loss_sum[0, 0] / (B * 2 * DIM)
    return (loss, pred)


# speedup vs baseline: 64.1170x; 1.7501x over previous
import functools

import jax
import jax.numpy as jnp
from jax import lax
from jax.experimental import pallas as pl
from jax.experimental.pallas import tpu as pltpu
from jax.experimental.pallas import tpu_sc as plsc

B, L, DIM = 16384, 20, 128
ZCH_0, ZCH_1 = 8, 16
NBINS = ZCH_0 + ZCH_1

NC, NS, LANES = 2, 16, 16
NW = NC * NS
BPT = B // NW
GROUPS = BPT // LANES

BLK = 2048


def _hist_body(ids0_hbm, ids1_hbm, cnt_hbm, ids0_v, ids1_v, cnt_v):
    wid = lax.axis_index("s") * NC + lax.axis_index("c")
    base = wid * BPT

    pltpu.sync_copy(ids0_hbm.at[:, pl.ds(base, BPT)], ids0_v)
    pltpu.sync_copy(ids1_hbm.at[:, pl.ds(base, BPT)], ids1_v)

    iota = lax.iota(jnp.int32, LANES)
    zeros = jnp.zeros((LANES,), jnp.float32)
    ones = zeros + 1.0

    def zcol(i, _):
        for r in range(NBINS):
            cnt_v[r, pl.ds(i * LANES, LANES)] = zeros
        return 0

    lax.fori_loop(0, BPT // LANES, zcol, 0)

    def group(g, _):
        lbag = g * LANES + iota
        col = g * LANES
        for l in range(L):
            g0 = ids0_v[l, pl.ds(col, LANES)]
            g1 = ids1_v[l, pl.ds(col, LANES)]
            e0 = g0 & (ZCH_0 - 1)
            e1 = (g1 & (ZCH_1 - 1)) + ZCH_0
            plsc.addupdate_scatter(cnt_v, [e0, lbag], ones)
            plsc.addupdate_scatter(cnt_v, [e1, lbag], ones)
        return 0

    lax.fori_loop(0, GROUPS, group, 0)

    pltpu.sync_copy(cnt_v, cnt_hbm.at[:, pl.ds(base, BPT)])


@functools.lru_cache(maxsize=None)
def _hist():
    return pl.kernel(
        _hist_body,
        out_type=jax.ShapeDtypeStruct((NBINS, B), jnp.float32),
        mesh=plsc.VectorSubcoreMesh(
            core_axis_name="c", subcore_axis_name="s", num_cores=NC, num_subcores=NS
        ),
        compiler_params=pltpu.CompilerParams(needs_layout_passes=False),
        scratch_types=[
            pltpu.VMEM((L, BPT), jnp.int32),
            pltpu.VMEM((L, BPT), jnp.int32),
            pltpu.VMEM((NBINS, BPT), jnp.float32),
        ],
    )


def _mm_body(ct_ref, t0_ref, t1_ref, out_ref, loss_ref):
    i = pl.program_id(0)
    ct = ct_ref[...]
    zz = jnp.zeros((ZCH_0, DIM), jnp.float32)
    w = jnp.concatenate(
        [
            jnp.concatenate([t0_ref[...], zz], axis=1),
            jnp.concatenate([jnp.zeros((ZCH_1, DIM), jnp.float32), t1_ref[...]], axis=1),
        ],
        axis=0,
    )
    ct_bf = ct.astype(jnp.bfloat16)
    w_hi = w.astype(jnp.bfloat16)
    w_lo = (w - w_hi.astype(jnp.float32)).astype(jnp.bfloat16)
    dn = (((0,), (0,)), ((), ()))
    p = lax.dot_general(ct_bf, w_hi, dn, preferred_element_type=jnp.float32)
    p = p + lax.dot_general(ct_bf, w_lo, dn, preferred_element_type=jnp.float32)
    out_ref[...] = p

    @pl.when(i == 0)
    def _():
        loss_ref[0, 0] = 0.0

    loss_ref[0, 0] += jnp.sum(p)


def _pool_matmul(ct, table_0, table_1):
    return pl.pallas_call(
        _mm_body,
        grid=(B // BLK,),
        in_specs=[
            pl.BlockSpec((NBINS, BLK), lambda i: (0, i)),
            pl.BlockSpec((ZCH_0, DIM), lambda i: (0, 0)),
            pl.BlockSpec((ZCH_1, DIM), lambda i: (0, 0)),
        ],
        out_specs=[
            pl.BlockSpec((BLK, 2 * DIM), lambda i: (i, 0)),
            pl.BlockSpec((1, 1), lambda i: (0, 0), memory_space=pltpu.SMEM),
        ],
        out_shape=[
            jax.ShapeDtypeStruct((B, 2 * DIM), jnp.float32),
            jax.ShapeDtypeStruct((1, 1), jnp.float32),
        ],
    )(ct, table_0, table_1)


@jax.jit
def kernel(ids_0, ids_1, table_0, table_1):
    ids0_t = ids_0.astype(jnp.int32).T
    ids1_t = ids_1.astype(jnp.int32).T
    ct = _hist()(ids0_t, ids1_t)
    pred, loss_sum = _pool_matmul(ct, table_0, table_1)
    loss = loss_sum[0, 0] / (B * 2 * DIM)
    return (loss, pred)
